# Initial kernel scaffold; baseline (speedup 1.0000x reference)
import jax
import jax.numpy as jnp
from jax.experimental import pallas as pl


def kernel(cls_ids, bbox_ids, cls_table, cx_table, cy_table, w_table, h_table):
    raise NotImplementedError("write your pallas kernel here")


# speedup vs baseline: 17.9196x; 17.9196x over previous
import functools

import jax
import jax.numpy as jnp
from jax import lax
from jax.experimental import pallas as pl
from jax.experimental.pallas import tpu as pltpu
from jax.experimental.pallas import tpu_sc as plsc

B, T = 4096, 200
BT = B * T
CLS_DIM, BBOX_DIM = 64, 16
OUT_DIM = CLS_DIM + 4 * BBOX_DIM

NC, NS = 2, 16
NW = NC * NS
TOK_PER_W = BT // NW
CHUNK = 128
N_CHUNKS = TOK_PER_W // CHUNK

_mesh = plsc.VectorSubcoreMesh(core_axis_name="c", subcore_axis_name="s")


@functools.partial(
    pl.kernel,
    out_type=jax.ShapeDtypeStruct((BT, OUT_DIM), jnp.float32),
    mesh=_mesh,
    scratch_types=[
        pltpu.VMEM((CHUNK,), jnp.int32),
        pltpu.VMEM((4, CHUNK), jnp.int32),
        pltpu.VMEM((CHUNK, CLS_DIM), jnp.float32),
        pltpu.VMEM((4, CHUNK, BBOX_DIM), jnp.float32),
        pltpu.SemaphoreType.DMA,
        pltpu.SemaphoreType.DMA,
    ],
    compiler_params=pltpu.CompilerParams(use_tc_tiling_on_sc=False),
)
def _emb_lookup(cls_idx_hbm, bbox_idx_hbm, cls_tab, cx_tab, cy_tab, w_tab,
                h_tab, out_hbm, cls_idx_v, bbox_idx_v, cls_rows_v,
                bbox_rows_v, gsem, wsem):
    wid = lax.axis_index("s") * NC + lax.axis_index("c")
    w_base = wid * TOK_PER_W
    bbox_tabs = (cx_tab, cy_tab, w_tab, h_tab)

    def chunk_body(c, carry):
        base = w_base + c * CHUNK
        pltpu.sync_copy(cls_idx_hbm.at[pl.ds(base, CHUNK)], cls_idx_v)
        pltpu.sync_copy(bbox_idx_hbm.at[:, pl.ds(base, CHUNK)], bbox_idx_v)
        gathers = [pltpu.async_copy(cls_tab.at[cls_idx_v], cls_rows_v, gsem)]
        for j, tab in enumerate(bbox_tabs):
            gathers.append(
                pltpu.async_copy(tab.at[bbox_idx_v.at[j]], bbox_rows_v.at[j],
                                 gsem))
        for g in gathers:
            g.wait()
        writes = [
            pltpu.async_copy(
                cls_rows_v, out_hbm.at[pl.ds(base, CHUNK), pl.ds(0, CLS_DIM)],
                wsem)
        ]
        for j in range(4):
            writes.append(
                pltpu.async_copy(
                    bbox_rows_v.at[j],
                    out_hbm.at[pl.ds(base, CHUNK),
                               pl.ds(CLS_DIM + j * BBOX_DIM, BBOX_DIM)],
                    wsem))
        for w in writes:
            w.wait()
        return carry

    lax.fori_loop(0, N_CHUNKS, chunk_body, 0)


def kernel(cls_ids, bbox_ids, cls_table, cx_table, cy_table, w_table, h_table):
    cls_flat = cls_ids.reshape(BT).astype(jnp.int32)
    bbox_t = jnp.transpose(bbox_ids.reshape(BT, 4)).astype(jnp.int32)
    out = _emb_lookup(cls_flat, bbox_t, cls_table, cx_table, cy_table,
                      w_table, h_table)
    return out.reshape(B, T, OUT_DIM)


# speedup vs baseline: 25.9692x; 1.4492x over previous
import functools

import jax
import jax.numpy as jnp
from jax import lax
from jax.experimental import pallas as pl
from jax.experimental.pallas import tpu as pltpu
from jax.experimental.pallas import tpu_sc as plsc

B, T = 4096, 200
BT = B * T
CLS_DIM, BBOX_DIM = 64, 16
OUT_DIM = CLS_DIM + 4 * BBOX_DIM

NC, NS = 2, 16
NW = NC * NS
TOK_PER_W = BT // NW
CHUNK = 128
N_CHUNKS = TOK_PER_W // CHUNK
NBUF = 4

_mesh = plsc.VectorSubcoreMesh(core_axis_name="c", subcore_axis_name="s")


@functools.partial(
    pl.kernel,
    out_type=jax.ShapeDtypeStruct((BT, OUT_DIM), jnp.float32),
    mesh=_mesh,
    scratch_types=[
        pltpu.VMEM((NBUF, CHUNK), jnp.int32),
        pltpu.VMEM((NBUF, 4, CHUNK), jnp.int32),
        pltpu.VMEM((NBUF, CHUNK, CLS_DIM), jnp.float32),
        pltpu.VMEM((NBUF, 4, CHUNK, BBOX_DIM), jnp.float32),
    ] + [pltpu.SemaphoreType.DMA] * (2 * NBUF),
    compiler_params=pltpu.CompilerParams(use_tc_tiling_on_sc=False),
)
def _emb_lookup(cls_idx_hbm, bbox_idx_hbm, cls_tab, cx_tab, cy_tab, w_tab,
                h_tab, out_hbm, cls_idx_v, bbox_idx_v, cls_rows_v,
                bbox_rows_v, *sems):
    gsems, wsems = sems[:NBUF], sems[NBUF:]
    wid = lax.axis_index("s") * NC + lax.axis_index("c")
    w_base = wid * TOK_PER_W
    bbox_tabs = (cx_tab, cy_tab, w_tab, h_tab)

    def load_idx(b, c):
        base = w_base + c * CHUNK
        pltpu.sync_copy(cls_idx_hbm.at[pl.ds(base, CHUNK)], cls_idx_v.at[b])
        pltpu.sync_copy(bbox_idx_hbm.at[:, pl.ds(base, CHUNK)],
                        bbox_idx_v.at[b])

    def gather_copies(b):
        cps = [pltpu.make_async_copy(cls_tab.at[cls_idx_v.at[b]],
                                     cls_rows_v.at[b], gsems[b])]
        for j, tab in enumerate(bbox_tabs):
            cps.append(pltpu.make_async_copy(tab.at[bbox_idx_v.at[b, j]],
                                             bbox_rows_v.at[b, j], gsems[b]))
        return cps

    def write_copies(b, c):
        base = w_base + c * CHUNK
        cps = [pltpu.make_async_copy(
            cls_rows_v.at[b],
            out_hbm.at[pl.ds(base, CHUNK), pl.ds(0, CLS_DIM)], wsems[b])]
        for j in range(4):
            cps.append(pltpu.make_async_copy(
                bbox_rows_v.at[b, j],
                out_hbm.at[pl.ds(base, CHUNK),
                           pl.ds(CLS_DIM + j * BBOX_DIM, BBOX_DIM)],
                wsems[b]))
        return cps

    for b in range(NBUF):
        load_idx(b, b)
        for cp in gather_copies(b):
            cp.start()

    @pl.loop(0, N_CHUNKS, step=NBUF)
    def _(c0):
        for b in range(NBUF):
            c = c0 + b
            for cp in gather_copies(b):
                cp.wait()
            for cp in write_copies(b, c):
                cp.start()

            @pl.when(c + NBUF < N_CHUNKS)
            def _():
                for cp in write_copies(b, c):
                    cp.wait()
                load_idx(b, c + NBUF)
                for cp in gather_copies(b):
                    cp.start()

    for b in range(NBUF):
        for cp in write_copies(b, N_CHUNKS - NBUF + b):
            cp.wait()


def kernel(cls_ids, bbox_ids, cls_table, cx_table, cy_table, w_table, h_table):
    cls_flat = cls_ids.reshape(BT).astype(jnp.int32)
    bbox_t = jnp.transpose(bbox_ids.reshape(BT, 4)).astype(jnp.int32)
    out = _emb_lookup(cls_flat, bbox_t, cls_table, cx_table, cy_table,
                      w_table, h_table)
    return out.reshape(B, T, OUT_DIM)


# speedup vs baseline: 26.0122x; 1.0017x over previous
import functools

import jax
import jax.numpy as jnp
from jax import lax
from jax.experimental import pallas as pl
from jax.experimental.pallas import tpu as pltpu
from jax.experimental.pallas import tpu_sc as plsc

B, T = 4096, 200
BT = B * T
CLS_DIM, BBOX_DIM = 64, 16
OUT_DIM = CLS_DIM + 4 * BBOX_DIM

NC, NS = 2, 16
NW = NC * NS
TOK_PER_W = BT // NW
CHUNK = 128
N_CHUNKS = TOK_PER_W // CHUNK
NBUF = 5

_mesh = plsc.VectorSubcoreMesh(core_axis_name="c", subcore_axis_name="s")


@functools.partial(
    pl.kernel,
    out_type=jax.ShapeDtypeStruct((BT, OUT_DIM), jnp.float32),
    mesh=_mesh,
    scratch_types=[
        pltpu.VMEM((NBUF, CHUNK), jnp.int32),
        pltpu.VMEM((NBUF, 4, CHUNK), jnp.int32),
        pltpu.VMEM((NBUF, CHUNK, CLS_DIM), jnp.float32),
        pltpu.VMEM((NBUF, 4, CHUNK, BBOX_DIM), jnp.float32),
    ] + [pltpu.SemaphoreType.DMA] * (2 * NBUF),
    compiler_params=pltpu.CompilerParams(use_tc_tiling_on_sc=False),
)
def _emb_lookup(cls_idx_hbm, bbox_idx_hbm, cls_tab, cx_tab, cy_tab, w_tab,
                h_tab, out_hbm, cls_idx_v, bbox_idx_v, cls_rows_v,
                bbox_rows_v, *sems):
    gsems, wsems = sems[:NBUF], sems[NBUF:]
    wid = lax.axis_index("s") * NC + lax.axis_index("c")
    w_base = wid * TOK_PER_W
    bbox_tabs = (cx_tab, cy_tab, w_tab, h_tab)

    def load_idx(b, c):
        base = w_base + c * CHUNK
        pltpu.sync_copy(cls_idx_hbm.at[pl.ds(base, CHUNK)], cls_idx_v.at[b])
        pltpu.sync_copy(bbox_idx_hbm.at[:, pl.ds(base, CHUNK)],
                        bbox_idx_v.at[b])

    def gather_copies(b):
        cps = [pltpu.make_async_copy(cls_tab.at[cls_idx_v.at[b]],
                                     cls_rows_v.at[b], gsems[b])]
        for j, tab in enumerate(bbox_tabs):
            cps.append(pltpu.make_async_copy(tab.at[bbox_idx_v.at[b, j]],
                                             bbox_rows_v.at[b, j], gsems[b]))
        return cps

    def write_copies(b, c):
        base = w_base + c * CHUNK
        cps = [pltpu.make_async_copy(
            cls_rows_v.at[b],
            out_hbm.at[pl.ds(base, CHUNK), pl.ds(0, CLS_DIM)], wsems[b])]
        for j in range(4):
            cps.append(pltpu.make_async_copy(
                bbox_rows_v.at[b, j],
                out_hbm.at[pl.ds(base, CHUNK),
                           pl.ds(CLS_DIM + j * BBOX_DIM, BBOX_DIM)],
                wsems[b]))
        return cps

    for b in range(NBUF):
        load_idx(b, b)
        for cp in gather_copies(b):
            cp.start()

    @pl.loop(0, N_CHUNKS, step=NBUF)
    def _(c0):
        for b in range(NBUF):
            c = c0 + b
            for cp in gather_copies(b):
                cp.wait()
            for cp in write_copies(b, c):
                cp.start()

            @pl.when(c + NBUF < N_CHUNKS)
            def _():
                for cp in write_copies(b, c):
                    cp.wait()
                load_idx(b, c + NBUF)
                for cp in gather_copies(b):
                    cp.start()

    for b in range(NBUF):
        for cp in write_copies(b, N_CHUNKS - NBUF + b):
            cp.wait()


def kernel(cls_ids, bbox_ids, cls_table, cx_table, cy_table, w_table, h_table):
    cls_flat = cls_ids.reshape(BT).astype(jnp.int32)
    bbox_t = jnp.transpose(bbox_ids.reshape(BT, 4)).astype(jnp.int32)
    out = _emb_lookup(cls_flat, bbox_t, cls_table, cx_table, cy_table,
                      w_table, h_table)
    return out.reshape(B, T, OUT_DIM)
